# baseline (speedup 1.0000x reference)
import functools

import jax
import jax.numpy as jnp
from jax import lax
from jax.experimental import pallas as pl
from jax.experimental.pallas import tpu as pltpu
from jax.experimental.pallas import tpu_sc as plsc

_B = 16384
_H = 200
_D = 32
_L = 16

_NW = 32
_RW = _B // _NW
_CB = 8
_NCH = _RW // _CB
_H0 = 128
_H1 = _H - _H0

_mesh = plsc.VectorSubcoreMesh(core_axis_name="c", subcore_axis_name="s")


def _pool_sums(x0, x1, emb):
    @functools.partial(
        pl.kernel,
        out_type=jax.ShapeDtypeStruct((_B, _D), jnp.float32),
        mesh=_mesh,
        scratch_types=[
            pltpu.VMEM((_CB, _H0), jnp.int32),
            pltpu.VMEM((_CB, _H0), jnp.int32),
            pltpu.VMEM((_CB, _H1), jnp.int32),
            pltpu.VMEM((_CB, _H1), jnp.int32),
            pltpu.VMEM((_CB * _H, _D), jnp.float32),
            pltpu.VMEM((_CB * _H, _D), jnp.float32),
            pltpu.VMEM((_RW, _D), jnp.float32),
            pltpu.SemaphoreType.DMA,
            pltpu.SemaphoreType.DMA,
            pltpu.SemaphoreType.DMA,
            pltpu.SemaphoreType.DMA,
        ],
        compiler_params=pltpu.CompilerParams(use_tc_tiling_on_sc=False),
    )
    def k(x0_hbm, x1_hbm, emb_hbm, out_hbm, ia_a, ia_b, ib_a, ib_b,
          rows_a, rows_b, out_v, isem_a, isem_b, gsem_a, gsem_b):
        ias = (ia_a, ia_b)
        ibs = (ib_a, ib_b)
        rows = (rows_a, rows_b)
        isems = (isem_a, isem_b)
        gsems = (gsem_a, gsem_b)

        wid = lax.axis_index("s") * 2 + lax.axis_index("c")
        row0 = wid * _RW

        class _IdxPair:
            def __init__(self, g, p):
                sl = pl.ds(row0 + g * _CB, _CB)
                self.c0 = pltpu.make_async_copy(
                    x0_hbm.at[sl], ias[p], isems[p])
                self.c1 = pltpu.make_async_copy(
                    x1_hbm.at[sl], ibs[p], isems[p])

            def start(self):
                self.c0.start()
                self.c1.start()

            def wait(self):
                self.c0.wait()
                self.c1.wait()

        idx_cp = _IdxPair

        def gathers_start(p):
            for e in range(_CB):
                pltpu.make_async_copy(
                    emb_hbm.at[ias[p].at[e]],
                    rows[p].at[pl.ds(e * _H, _H0)], gsems[p]).start()
                pltpu.make_async_copy(
                    emb_hbm.at[ibs[p].at[e]],
                    rows[p].at[pl.ds(e * _H + _H0, _H1)], gsems[p]).start()

        def gathers_wait(p):
            for e in range(_CB):
                pltpu.make_async_copy(
                    emb_hbm.at[ias[p].at[e]],
                    rows[p].at[pl.ds(e * _H, _H0)], gsems[p]).wait()
                pltpu.make_async_copy(
                    emb_hbm.at[ibs[p].at[e]],
                    rows[p].at[pl.ds(e * _H + _H0, _H1)], gsems[p]).wait()

        def compute(g, p):
            r_v = rows[p]
            for e in range(_CB):
                def body(t, acc, e=e):
                    a0, a1, b0, b1 = acc
                    base = e * _H + t * 8
                    for u in range(0, 8, 2):
                        a0 = a0 + r_v[base + u, pl.ds(0, _L)]
                        a1 = a1 + r_v[base + u, pl.ds(_L, _L)]
                        b0 = b0 + r_v[base + u + 1, pl.ds(0, _L)]
                        b1 = b1 + r_v[base + u + 1, pl.ds(_L, _L)]
                    return (a0, a1, b0, b1)
                z = jnp.zeros((_L,), jnp.float32)
                a0, a1, b0, b1 = lax.fori_loop(0, _H // 8, body, (z, z, z, z))
                out_v[g * _CB + e, pl.ds(0, _L)] = a0 + b0
                out_v[g * _CB + e, pl.ds(_L, _L)] = a1 + b1

        c0 = idx_cp(0, 0)
        c0.start()
        c0.wait()
        gathers_start(0)
        idx_cp(1, 1).start()

        @pl.loop(0, _NCH, step=2)
        def _(g):
            idx_cp(g + 1, 1).wait()
            gathers_start(1)
            gathers_wait(0)
            @pl.when(g + 2 < _NCH)
            def _():
                idx_cp(g + 2, 0).start()
            compute(g, 0)
            @pl.when(g + 2 < _NCH)
            def _():
                idx_cp(g + 2, 0).wait()
                gathers_start(0)
            gathers_wait(1)
            @pl.when(g + 3 < _NCH)
            def _():
                idx_cp(g + 3, 1).start()
            compute(g + 1, 1)

        pltpu.sync_copy(out_v, out_hbm.at[pl.ds(wid * _RW, _RW)])

    return k(x0, x1, emb)


def _relayout(emb_t):
    v = emb_t.shape[1]
    grid = (v + 1023) // 1024

    def body(in_ref, o_ref):
        o_ref[:, 0:_D] = jnp.transpose(in_ref[...])

    return pl.pallas_call(
        body,
        grid=(grid,),
        in_specs=[pl.BlockSpec((_D, 1024), lambda i: (0, i))],
        out_specs=pl.BlockSpec((1024, 128), lambda i: (i, 0)),
        out_shape=jax.ShapeDtypeStruct((grid * 1024, 128), jnp.float32),
    )(emb_t)


def _mlp(sums, w1t, b1, w2t, b2):
    def body(s_ref, w1_ref, b1_ref, w2_ref, b2_ref, o_ref):
        mean = s_ref[...] * (1.0 / _H)
        h = jnp.dot(mean, w1_ref[...], preferred_element_type=jnp.float32)
        h = jax.nn.sigmoid(h + b1_ref[...])
        t = jnp.dot(h, w2_ref[...], preferred_element_type=jnp.float32)
        o_ref[...] = jax.nn.sigmoid(t + b2_ref[...])

    return pl.pallas_call(
        body,
        out_shape=jax.ShapeDtypeStruct((_B, 1), jnp.float32),
    )(sums, w1t, b1, w2t, b2)


def kernel(x, emb, W1, b1, W2, b2):
    x4 = x.astype(jnp.int32) * 4
    x0 = lax.slice(x4, (0, 0), (_B, _H0))
    x1 = lax.slice(x4, (0, _H0), (_B, _H))
    big = _relayout(emb.T)
    emb_lin = big.reshape(big.shape[0] * (128 // _D), _D)
    sums = _pool_sums(x0, x1, emb_lin)
    return _mlp(sums, W1.T, b1.reshape(1, 16), W2.T, b2.reshape(1, 1))


# speedup vs baseline: 1.1049x; 1.1049x over previous
import functools

import jax
import jax.numpy as jnp
from jax import lax
from jax.experimental import pallas as pl
from jax.experimental.pallas import tpu as pltpu
from jax.experimental.pallas import tpu_sc as plsc

_B = 16384
_H = 200
_D = 32
_L = 16

_NW = 32
_RW = _B // _NW
_CB = 8
_NCH = _RW // _CB
_H0 = 128
_H1 = _H - _H0

_mesh = plsc.VectorSubcoreMesh(core_axis_name="c", subcore_axis_name="s")


def _pool_sums(x0, x1, emb):
    @functools.partial(
        pl.kernel,
        out_type=jax.ShapeDtypeStruct((_B, _D), jnp.float32),
        mesh=_mesh,
        scratch_types=[
            pltpu.VMEM((_CB, _H0), jnp.int32),
            pltpu.VMEM((_CB, _H0), jnp.int32),
            pltpu.VMEM((_CB, _H1), jnp.int32),
            pltpu.VMEM((_CB, _H1), jnp.int32),
            pltpu.VMEM((_CB * _H, _D), jnp.float32),
            pltpu.VMEM((_CB * _H, _D), jnp.float32),
            pltpu.VMEM((_RW, _D), jnp.float32),
            pltpu.SemaphoreType.DMA,
            pltpu.SemaphoreType.DMA,
            pltpu.SemaphoreType.DMA,
            pltpu.SemaphoreType.DMA,
        ],
        compiler_params=pltpu.CompilerParams(use_tc_tiling_on_sc=False),
    )
    def k(x0_hbm, x1_hbm, emb_hbm, out_hbm, ia_a, ia_b, ib_a, ib_b,
          rows_a, rows_b, out_v, isem_a, isem_b, gsem_a, gsem_b):
        ias = (ia_a, ia_b)
        ibs = (ib_a, ib_b)
        rows = (rows_a, rows_b)
        isems = (isem_a, isem_b)
        gsems = (gsem_a, gsem_b)

        wid = lax.axis_index("s") * 2 + lax.axis_index("c")
        row0 = wid * _RW

        class _IdxPair:
            def __init__(self, g, p):
                sl = pl.ds(row0 + g * _CB, _CB)
                self.c0 = pltpu.make_async_copy(
                    x0_hbm.at[sl], ias[p], isems[p])
                self.c1 = pltpu.make_async_copy(
                    x1_hbm.at[sl], ibs[p], isems[p])

            def start(self):
                self.c0.start()
                self.c1.start()

            def wait(self):
                self.c0.wait()
                self.c1.wait()

        idx_cp = _IdxPair

        def gathers_start(p):
            for e in range(_CB):
                pltpu.make_async_copy(
                    emb_hbm.at[ias[p].at[e]],
                    rows[p].at[pl.ds(e * _H, _H0)], gsems[p]).start()
                pltpu.make_async_copy(
                    emb_hbm.at[ibs[p].at[e]],
                    rows[p].at[pl.ds(e * _H + _H0, _H1)], gsems[p]).start()

        def gathers_wait(p):
            for e in range(_CB):
                pltpu.make_async_copy(
                    emb_hbm.at[ias[p].at[e]],
                    rows[p].at[pl.ds(e * _H, _H0)], gsems[p]).wait()
                pltpu.make_async_copy(
                    emb_hbm.at[ibs[p].at[e]],
                    rows[p].at[pl.ds(e * _H + _H0, _H1)], gsems[p]).wait()

        def compute(g, p):
            r_v = rows[p]
            for e in range(_CB):
                def body(t, acc, e=e):
                    a0, a1, b0, b1 = acc
                    base = e * _H + t * 8
                    for u in range(0, 8, 2):
                        a0 = a0 + r_v[base + u, pl.ds(0, _L)]
                        a1 = a1 + r_v[base + u, pl.ds(_L, _L)]
                        b0 = b0 + r_v[base + u + 1, pl.ds(0, _L)]
                        b1 = b1 + r_v[base + u + 1, pl.ds(_L, _L)]
                    return (a0, a1, b0, b1)
                z = jnp.zeros((_L,), jnp.float32)
                a0, a1, b0, b1 = lax.fori_loop(0, _H // 8, body, (z, z, z, z))
                out_v[g * _CB + e, pl.ds(0, _L)] = a0 + b0
                out_v[g * _CB + e, pl.ds(_L, _L)] = a1 + b1

        c0 = idx_cp(0, 0)
        c0.start()
        c0.wait()
        gathers_start(0)
        idx_cp(1, 1).start()

        @pl.loop(0, _NCH, step=2)
        def _(g):
            idx_cp(g + 1, 1).wait()
            gathers_start(1)
            gathers_wait(0)
            @pl.when(g + 2 < _NCH)
            def _():
                idx_cp(g + 2, 0).start()
            compute(g, 0)
            @pl.when(g + 2 < _NCH)
            def _():
                idx_cp(g + 2, 0).wait()
                gathers_start(0)
            gathers_wait(1)
            @pl.when(g + 3 < _NCH)
            def _():
                idx_cp(g + 3, 1).start()
            compute(g + 1, 1)

        pltpu.sync_copy(out_v, out_hbm.at[pl.ds(wid * _RW, _RW)])

    return k(x0, x1, emb)


_P = 250880


def _relayout(emb):
    nb = _P // 1024
    last = (emb.shape[0] - 1) // 1024

    def body(a0, a1, a2, a3, o_ref):
        o_ref[...] = jnp.concatenate(
            [a0[...], a1[...], a2[...], a3[...]], axis=1)

    return pl.pallas_call(
        body,
        grid=(nb,),
        in_specs=[pl.BlockSpec(
            (1024, _D), (lambda i, a=a: (jnp.minimum(nb * a + i, last), 0)))
            for a in range(4)],
        out_specs=pl.BlockSpec((1024, 128), lambda i: (i, 0)),
        out_shape=jax.ShapeDtypeStruct((_P, 128), jnp.float32),
    )(emb, emb, emb, emb)


def _mlp(sums, w1t, b1, w2t, b2):
    def body(s_ref, w1_ref, b1_ref, w2_ref, b2_ref, o_ref):
        mean = s_ref[...] * (1.0 / _H)
        h = jnp.dot(mean, w1_ref[...], preferred_element_type=jnp.float32)
        h = jax.nn.sigmoid(h + b1_ref[...])
        t = jnp.dot(h, w2_ref[...], preferred_element_type=jnp.float32)
        o_ref[...] = jax.nn.sigmoid(t + b2_ref[...])

    return pl.pallas_call(
        body,
        out_shape=jax.ShapeDtypeStruct((_B, 1), jnp.float32),
    )(sums, w1t, b1, w2t, b2)


def kernel(x, emb, W1, b1, W2, b2):
    xi = x.astype(jnp.int32)
    xk = 4 * (xi % _P) + xi // _P
    x0 = lax.slice(xk, (0, 0), (_B, _H0))
    x1 = lax.slice(xk, (0, _H0), (_B, _H))
    emb_lin = _relayout(emb).reshape(4 * _P, _D)
    sums = _pool_sums(x0, x1, emb_lin)
    return _mlp(sums, W1.T, b1.reshape(1, 16), W2.T, b2.reshape(1, 1))


# speedup vs baseline: 1.7053x; 1.5434x over previous
import functools

import jax
import jax.numpy as jnp
from jax import lax
from jax.experimental import pallas as pl
from jax.experimental.pallas import tpu as pltpu
from jax.experimental.pallas import tpu_sc as plsc

_B = 16384
_H = 200
_D = 32
_L = 16

_NW = 32
_RW = _B // _NW
_CB = 8
_NCH = _RW // _CB
_H0 = 128
_H1 = _H - _H0

_mesh = plsc.VectorSubcoreMesh(core_axis_name="c", subcore_axis_name="s")


def _pool_sums(x0, x1, emb):
    @functools.partial(
        pl.kernel,
        out_type=jax.ShapeDtypeStruct((_B, _D), jnp.float32),
        mesh=_mesh,
        scratch_types=[
            pltpu.VMEM((_CB, _H0), jnp.int32),
            pltpu.VMEM((_CB, _H0), jnp.int32),
            pltpu.VMEM((_CB, _H1), jnp.int32),
            pltpu.VMEM((_CB, _H1), jnp.int32),
            pltpu.VMEM((_CB * _H, _D), jnp.float32),
            pltpu.VMEM((_CB * _H, _D), jnp.float32),
            pltpu.VMEM((_RW, _D), jnp.float32),
            pltpu.SemaphoreType.DMA,
            pltpu.SemaphoreType.DMA,
            pltpu.SemaphoreType.DMA,
            pltpu.SemaphoreType.DMA,
        ],
        compiler_params=pltpu.CompilerParams(use_tc_tiling_on_sc=False),
    )
    def k(x0_hbm, x1_hbm, emb_hbm, out_hbm, ia_a, ia_b, ib_a, ib_b,
          rows_a, rows_b, out_v, isem_a, isem_b, gsem_a, gsem_b):
        ias = (ia_a, ia_b)
        ibs = (ib_a, ib_b)
        rows = (rows_a, rows_b)
        isems = (isem_a, isem_b)
        gsems = (gsem_a, gsem_b)

        wid = lax.axis_index("s") * 2 + lax.axis_index("c")
        row0 = wid * _RW

        class _IdxPair:
            def __init__(self, g, p):
                sl = pl.ds(row0 + g * _CB, _CB)
                self.c0 = pltpu.make_async_copy(
                    x0_hbm.at[sl], ias[p], isems[p])
                self.c1 = pltpu.make_async_copy(
                    x1_hbm.at[sl], ibs[p], isems[p])

            def start(self):
                self.c0.start()
                self.c1.start()

            def wait(self):
                self.c0.wait()
                self.c1.wait()

        idx_cp = _IdxPair

        def gathers_start(p):
            for e in range(_CB):
                pltpu.make_async_copy(
                    emb_hbm.at[ias[p].at[e]],
                    rows[p].at[pl.ds(e * _H, _H0)], gsems[p]).start()
                pltpu.make_async_copy(
                    emb_hbm.at[ibs[p].at[e]],
                    rows[p].at[pl.ds(e * _H + _H0, _H1)], gsems[p]).start()

        def gathers_wait(p):
            for e in range(_CB):
                pltpu.make_async_copy(
                    emb_hbm.at[ias[p].at[e]],
                    rows[p].at[pl.ds(e * _H, _H0)], gsems[p]).wait()
                pltpu.make_async_copy(
                    emb_hbm.at[ibs[p].at[e]],
                    rows[p].at[pl.ds(e * _H + _H0, _H1)], gsems[p]).wait()

        def compute(g, p):
            r_v = rows[p]
            for e in range(_CB):
                def body(t, acc, e=e):
                    a0, a1, b0, b1 = acc
                    base = e * _H + t * 8
                    for u in range(0, 8, 2):
                        a0 = a0 + r_v[base + u, pl.ds(0, _L)]
                        a1 = a1 + r_v[base + u, pl.ds(_L, _L)]
                        b0 = b0 + r_v[base + u + 1, pl.ds(0, _L)]
                        b1 = b1 + r_v[base + u + 1, pl.ds(_L, _L)]
                    return (a0, a1, b0, b1)
                z = jnp.zeros((_L,), jnp.float32)
                a0, a1, b0, b1 = lax.fori_loop(0, _H // 8, body, (z, z, z, z))
                out_v[g * _CB + e, pl.ds(0, _L)] = a0 + b0
                out_v[g * _CB + e, pl.ds(_L, _L)] = a1 + b1

        c0 = idx_cp(0, 0)
        c0.start()
        c0.wait()
        gathers_start(0)
        idx_cp(1, 1).start()

        @pl.loop(0, _NCH, step=2)
        def _(g):
            idx_cp(g + 1, 1).wait()
            gathers_start(1)
            gathers_wait(0)
            @pl.when(g + 2 < _NCH)
            def _():
                idx_cp(g + 2, 0).start()
            compute(g, 0)
            @pl.when(g + 2 < _NCH)
            def _():
                idx_cp(g + 2, 0).wait()
                gathers_start(0)
            gathers_wait(1)
            @pl.when(g + 3 < _NCH)
            def _():
                idx_cp(g + 3, 1).start()
            compute(g + 1, 1)

        pltpu.sync_copy(out_v, out_hbm.at[pl.ds(wid * _RW, _RW)])

    return k(x0, x1, emb)


_P = 250880


def _relayout(emb_t):
    nb = _P // 1024
    last = (emb_t.shape[1] - 1) // 1024

    def body(a0, a1, a2, a3, o_ref):
        i0 = lax.broadcasted_iota(jnp.int32, (_D, _D), 0)
        i1 = lax.broadcasted_iota(jnp.int32, (_D, _D), 1)
        eye = (i0 == i1).astype(jnp.float32)
        parts = [jnp.einsum('dc,dj->cj', a[...], eye,
                            preferred_element_type=jnp.float32)
                 for a in (a0, a1, a2, a3)]
        o_ref[...] = jnp.concatenate(parts, axis=1)

    return pl.pallas_call(
        body,
        grid=(nb,),
        in_specs=[pl.BlockSpec(
            (_D, 1024), (lambda i, a=a: (0, jnp.minimum(nb * a + i, last))))
            for a in range(4)],
        out_specs=pl.BlockSpec((1024, 128), lambda i: (i, 0)),
        out_shape=jax.ShapeDtypeStruct((_P, 128), jnp.float32),
    )(emb_t, emb_t, emb_t, emb_t)


def _mlp(sums, w1t, b1, w2t, b2):
    def body(s_ref, w1_ref, b1_ref, w2_ref, b2_ref, o_ref):
        mean = s_ref[...] * (1.0 / _H)
        h = jnp.dot(mean, w1_ref[...], preferred_element_type=jnp.float32)
        h = jax.nn.sigmoid(h + b1_ref[...])
        t = jnp.dot(h, w2_ref[...], preferred_element_type=jnp.float32)
        o_ref[...] = jax.nn.sigmoid(t + b2_ref[...])

    return pl.pallas_call(
        body,
        out_shape=jax.ShapeDtypeStruct((_B, 1), jnp.float32),
    )(sums, w1t, b1, w2t, b2)


def kernel(x, emb, W1, b1, W2, b2):
    xi = x.astype(jnp.int32)
    xk = 4 * (xi % _P) + xi // _P
    x0 = lax.slice(xk, (0, 0), (_B, _H0))
    x1 = lax.slice(xk, (0, _H0), (_B, _H))
    emb_lin = _relayout(emb.T).reshape(4 * _P, _D)
    sums = _pool_sums(x0, x1, emb_lin)
    return _mlp(sums, W1.T, b1.reshape(1, 16), W2.T, b2.reshape(1, 1))
